# baseline (speedup 1.0000x reference)
import jax
import jax.numpy as jnp
from jax.experimental import pallas as pl
from jax.experimental.pallas import tpu as pltpu

_K = 8
_COL_BOX = _K
_COL_CTR = _K + 4
_COL_DELTA = _K + 5
_SHIFT_OFFSET = 0.5
_FPN_STRIDES = (8, 16)
_LANES = 128
_SLABS = 4


def _make_fused_kernel(r0, r1):
    def _fused(img_ref, mean_ref, std_ref, x0_ref, x1_ref, w_ref, sb_ref,
               s2_ref,
               img_out_ref, cls_ref, ctr_ref, delta_ref, score_ref, box_ref,
               shifts_ref):
        img_out_ref[...] = ((img_ref[...] - mean_ref[...])
                            * (1.0 / std_ref[...]))

        @pl.when(pl.program_id(1) == 0)
        def _head():
            shifts_ref[...] = s2_ref[...]
            w = w_ref[...]

            def do_level(x, o, rl):
                acc = jax.lax.dot_general(
                    x, w, (((0,), (0,)), ((), ())),
                    preferred_element_type=jnp.float32)
                full = acc + sb_ref[o:o + rl, :]
                cls = full[:, :_K]
                ctr = full[:, _COL_CTR:_COL_CTR + 1]
                cls_ref[o:o + rl, :] = cls
                ctr_ref[o:o + rl, :] = ctr
                delta_ref[o:o + rl, :] = full[:, _COL_DELTA:_COL_DELTA + 4]
                box_ref[o:o + rl, :] = full[:, _COL_BOX:_COL_BOX + 4]
                score_ref[o:o + rl, :] = jnp.sqrt(
                    jax.nn.sigmoid(cls) * jax.nn.sigmoid(ctr))

            do_level(x0_ref[0], 0, r0)
            do_level(x1_ref[0], r0, r1)

    return _fused


def _make_shift2(h, w, stride):
    ys = (jnp.arange(h, dtype=jnp.float32) + _SHIFT_OFFSET) * stride
    xs = (jnp.arange(w, dtype=jnp.float32) + _SHIFT_OFFSET) * stride
    yy, xx = jnp.meshgrid(ys, xs, indexing="ij")
    return jnp.stack([xx.reshape(-1), yy.reshape(-1)], axis=-1)


def kernel(images, feat0, feat1, pixel_mean, pixel_std, w_full, b_full):
    n, c, h, w = images.shape
    _, fc, h0, w0 = feat0.shape
    _, _, h1, w1 = feat1.shape
    r0, r1 = h0 * w0, h1 * w1
    r = r0 + r1
    m = n * r
    width = w_full.shape[1]

    x0 = feat0.reshape(n, fc, r0)
    x1 = feat1.reshape(n, fc, r1)

    shift_img = jnp.concatenate(
        [_make_shift2(h0, w0, _FPN_STRIDES[0]),
         _make_shift2(h1, w1, _FPN_STRIDES[1])], axis=0)
    shift4_img = jnp.concatenate([shift_img, shift_img], axis=1)
    sb = jnp.pad(shift4_img, ((0, 0), (_COL_BOX, _LANES - _COL_BOX - 4)))
    sb = sb + b_full

    outs = pl.pallas_call(
        _make_fused_kernel(r0, r1),
        out_shape=(
            jax.ShapeDtypeStruct((n, c, h, w), jnp.float32),
            jax.ShapeDtypeStruct((m, _K), jnp.float32),
            jax.ShapeDtypeStruct((m, 1), jnp.float32),
            jax.ShapeDtypeStruct((m, 4), jnp.float32),
            jax.ShapeDtypeStruct((m, _K), jnp.float32),
            jax.ShapeDtypeStruct((m, 4), jnp.float32),
            jax.ShapeDtypeStruct((m, 2), jnp.float32),
        ),
        grid=(n, _SLABS),
        in_specs=[
            pl.BlockSpec((1, c, h // _SLABS, w), lambda i, j: (i, 0, j, 0)),
            pl.BlockSpec((1, c, 1, 1), lambda i, j: (0, 0, 0, 0)),
            pl.BlockSpec((1, c, 1, 1), lambda i, j: (0, 0, 0, 0)),
            pl.BlockSpec((1, fc, r0), lambda i, j: (i, 0, 0)),
            pl.BlockSpec((1, fc, r1), lambda i, j: (i, 0, 0)),
            pl.BlockSpec((fc, width), lambda i, j: (0, 0)),
            pl.BlockSpec((r, _LANES), lambda i, j: (0, 0)),
            pl.BlockSpec((r, 2), lambda i, j: (0, 0)),
        ],
        out_specs=(
            pl.BlockSpec((1, c, h // _SLABS, w), lambda i, j: (i, 0, j, 0)),
            pl.BlockSpec((r, _K), lambda i, j: (i, 0)),
            pl.BlockSpec((r, 1), lambda i, j: (i, 0)),
            pl.BlockSpec((r, 4), lambda i, j: (i, 0)),
            pl.BlockSpec((r, _K), lambda i, j: (i, 0)),
            pl.BlockSpec((r, 4), lambda i, j: (i, 0)),
            pl.BlockSpec((r, 2), lambda i, j: (i, 0)),
        ),
        compiler_params=pltpu.CompilerParams(
            dimension_semantics=("parallel", "arbitrary")),
    )(images, pixel_mean.reshape(1, c, 1, 1), pixel_std.reshape(1, c, 1, 1),
      x0, x1, w_full, sb, shift_img)
    (images_norm, box_cls, box_ctr, box_delta,
     scores, boxes, shifts) = outs

    return images_norm, box_cls, box_ctr, box_delta, scores, boxes, shifts


# speedup vs baseline: 1.0946x; 1.0946x over previous
import jax
import jax.numpy as jnp
from jax.experimental import pallas as pl
from jax.experimental.pallas import tpu as pltpu

_K = 8
_COL_BOX = _K
_COL_CTR = _K + 4
_COL_DELTA = _K + 5
_SHIFT_OFFSET = 0.5
_FPN_STRIDES = (8, 16)
_LANES = 128


def _make_fused_kernel(r0, r1):
    def _fused(img_ref, mean_ref, std_ref, x0_ref, x1_ref, w_ref, sb_ref,
               s2_ref,
               img_out_ref, cls_ref, ctr_ref, delta_ref, score_ref, box_ref,
               shifts_ref):
        img_out_ref[...] = ((img_ref[...] - mean_ref[...])
                            * (1.0 / std_ref[...]))

        shifts_ref[...] = s2_ref[...]
        w = w_ref[...]

        def do_level(x, o, rl):
            acc = jax.lax.dot_general(
                x, w, (((0,), (0,)), ((), ())),
                preferred_element_type=jnp.float32)
            full = acc + sb_ref[o:o + rl, :]
            cls = full[:, :_K]
            ctr = full[:, _COL_CTR:_COL_CTR + 1]
            cls_ref[o:o + rl, :] = cls
            ctr_ref[o:o + rl, :] = ctr
            delta_ref[o:o + rl, :] = full[:, _COL_DELTA:_COL_DELTA + 4]
            box_ref[o:o + rl, :] = full[:, _COL_BOX:_COL_BOX + 4]
            score_ref[o:o + rl, :] = jnp.sqrt(
                jax.nn.sigmoid(cls) * jax.nn.sigmoid(ctr))

        do_level(x0_ref[0], 0, r0)
        do_level(x1_ref[0], r0, r1)

    return _fused


def _make_shift2(h, w, stride):
    ys = (jnp.arange(h, dtype=jnp.float32) + _SHIFT_OFFSET) * stride
    xs = (jnp.arange(w, dtype=jnp.float32) + _SHIFT_OFFSET) * stride
    yy, xx = jnp.meshgrid(ys, xs, indexing="ij")
    return jnp.stack([xx.reshape(-1), yy.reshape(-1)], axis=-1)


def kernel(images, feat0, feat1, pixel_mean, pixel_std, w_full, b_full):
    n, c, h, w = images.shape
    _, fc, h0, w0 = feat0.shape
    _, _, h1, w1 = feat1.shape
    r0, r1 = h0 * w0, h1 * w1
    r = r0 + r1
    m = n * r
    width = w_full.shape[1]

    x0 = feat0.reshape(n, fc, r0)
    x1 = feat1.reshape(n, fc, r1)

    shift_img = jnp.concatenate(
        [_make_shift2(h0, w0, _FPN_STRIDES[0]),
         _make_shift2(h1, w1, _FPN_STRIDES[1])], axis=0)
    shift4_img = jnp.concatenate([shift_img, shift_img], axis=1)
    sb = jnp.pad(shift4_img, ((0, 0), (_COL_BOX, _LANES - _COL_BOX - 4)))
    sb = sb + b_full

    outs = pl.pallas_call(
        _make_fused_kernel(r0, r1),
        out_shape=(
            jax.ShapeDtypeStruct((n, c, h, w), jnp.float32),
            jax.ShapeDtypeStruct((m, _K), jnp.float32),
            jax.ShapeDtypeStruct((m, 1), jnp.float32),
            jax.ShapeDtypeStruct((m, 4), jnp.float32),
            jax.ShapeDtypeStruct((m, _K), jnp.float32),
            jax.ShapeDtypeStruct((m, 4), jnp.float32),
            jax.ShapeDtypeStruct((m, 2), jnp.float32),
        ),
        grid=(n,),
        in_specs=[
            pl.BlockSpec((1, c, h, w), lambda i: (i, 0, 0, 0)),
            pl.BlockSpec((1, c, 1, 1), lambda i: (0, 0, 0, 0)),
            pl.BlockSpec((1, c, 1, 1), lambda i: (0, 0, 0, 0)),
            pl.BlockSpec((1, fc, r0), lambda i: (i, 0, 0)),
            pl.BlockSpec((1, fc, r1), lambda i: (i, 0, 0)),
            pl.BlockSpec((fc, width), lambda i: (0, 0)),
            pl.BlockSpec((r, _LANES), lambda i: (0, 0)),
            pl.BlockSpec((r, 2), lambda i: (0, 0)),
        ],
        out_specs=(
            pl.BlockSpec((1, c, h, w), lambda i: (i, 0, 0, 0)),
            pl.BlockSpec((r, _K), lambda i: (i, 0)),
            pl.BlockSpec((r, 1), lambda i: (i, 0)),
            pl.BlockSpec((r, 4), lambda i: (i, 0)),
            pl.BlockSpec((r, _K), lambda i: (i, 0)),
            pl.BlockSpec((r, 4), lambda i: (i, 0)),
            pl.BlockSpec((r, 2), lambda i: (i, 0)),
        ),
        compiler_params=pltpu.CompilerParams(dimension_semantics=("parallel",)),
    )(images, pixel_mean.reshape(1, c, 1, 1), pixel_std.reshape(1, c, 1, 1),
      x0, x1, w_full, sb, shift_img)
    (images_norm, box_cls, box_ctr, box_delta,
     scores, boxes, shifts) = outs

    return images_norm, box_cls, box_ctr, box_delta, scores, boxes, shifts


# speedup vs baseline: 1.2660x; 1.1566x over previous
import jax
import jax.numpy as jnp
from jax.experimental import pallas as pl
from jax.experimental.pallas import tpu as pltpu

_K = 8
_COL_BOX = _K
_COL_CTR = _K + 4
_COL_DELTA = _K + 5
_SHIFT_OFFSET = 0.5
_FPN_STRIDES = (8, 16)
_LANES = 128


def _make_fused_kernel(r0, r1):
    def _fused(img_ref, mean_ref, std_ref, x0_ref, x1_ref, w_ref, sb_ref,
               img_out_ref, cls_ref, ctr_ref, delta_ref, score_ref, box_ref):
        img_out_ref[...] = ((img_ref[...] - mean_ref[...])
                            * (1.0 / std_ref[...]))

        w = w_ref[...]

        def do_level(x, o, rl):
            acc = jax.lax.dot_general(
                x, w, (((0,), (0,)), ((), ())),
                preferred_element_type=jnp.float32)
            full = acc + sb_ref[o:o + rl, :]
            cls = full[:, :_K]
            ctr = full[:, _COL_CTR:_COL_CTR + 1]
            cls_ref[o:o + rl, :] = cls
            ctr_ref[o:o + rl, :] = ctr
            delta_ref[o:o + rl, :] = full[:, _COL_DELTA:_COL_DELTA + 4]
            box_ref[o:o + rl, :] = full[:, _COL_BOX:_COL_BOX + 4]
            score_ref[o:o + rl, :] = jnp.sqrt(
                jax.nn.sigmoid(cls) * jax.nn.sigmoid(ctr))

        do_level(x0_ref[0], 0, r0)
        do_level(x1_ref[0], r0, r1)

    return _fused


def _make_shift2(h, w, stride):
    ys = (jnp.arange(h, dtype=jnp.float32) + _SHIFT_OFFSET) * stride
    xs = (jnp.arange(w, dtype=jnp.float32) + _SHIFT_OFFSET) * stride
    yy, xx = jnp.meshgrid(ys, xs, indexing="ij")
    return jnp.stack([xx.reshape(-1), yy.reshape(-1)], axis=-1)


def kernel(images, feat0, feat1, pixel_mean, pixel_std, w_full, b_full):
    n, c, h, w = images.shape
    _, fc, h0, w0 = feat0.shape
    _, _, h1, w1 = feat1.shape
    r0, r1 = h0 * w0, h1 * w1
    r = r0 + r1
    m = n * r
    width = w_full.shape[1]

    x0 = feat0.reshape(n, fc, r0)
    x1 = feat1.reshape(n, fc, r1)

    shift_img = jnp.concatenate(
        [_make_shift2(h0, w0, _FPN_STRIDES[0]),
         _make_shift2(h1, w1, _FPN_STRIDES[1])], axis=0)
    shift4_img = jnp.concatenate([shift_img, shift_img], axis=1)
    sb = jnp.pad(shift4_img, ((0, 0), (_COL_BOX, _LANES - _COL_BOX - 4)))
    sb = sb + b_full

    outs = pl.pallas_call(
        _make_fused_kernel(r0, r1),
        out_shape=(
            jax.ShapeDtypeStruct((n, c, h, w), jnp.float32),
            jax.ShapeDtypeStruct((m, _K), jnp.float32),
            jax.ShapeDtypeStruct((m, 1), jnp.float32),
            jax.ShapeDtypeStruct((m, 4), jnp.float32),
            jax.ShapeDtypeStruct((m, _K), jnp.float32),
            jax.ShapeDtypeStruct((m, 4), jnp.float32),
        ),
        grid=(n,),
        in_specs=[
            pl.BlockSpec((1, c, h, w), lambda i: (i, 0, 0, 0)),
            pl.BlockSpec((1, c, 1, 1), lambda i: (0, 0, 0, 0)),
            pl.BlockSpec((1, c, 1, 1), lambda i: (0, 0, 0, 0)),
            pl.BlockSpec((1, fc, r0), lambda i: (i, 0, 0)),
            pl.BlockSpec((1, fc, r1), lambda i: (i, 0, 0)),
            pl.BlockSpec((fc, width), lambda i: (0, 0)),
            pl.BlockSpec((r, _LANES), lambda i: (0, 0)),
        ],
        out_specs=(
            pl.BlockSpec((1, c, h, w), lambda i: (i, 0, 0, 0)),
            pl.BlockSpec((r, _K), lambda i: (i, 0)),
            pl.BlockSpec((r, 1), lambda i: (i, 0)),
            pl.BlockSpec((r, 4), lambda i: (i, 0)),
            pl.BlockSpec((r, _K), lambda i: (i, 0)),
            pl.BlockSpec((r, 4), lambda i: (i, 0)),
        ),
        compiler_params=pltpu.CompilerParams(dimension_semantics=("parallel",)),
    )(images, pixel_mean.reshape(1, c, 1, 1), pixel_std.reshape(1, c, 1, 1),
      x0, x1, w_full, sb)
    images_norm, box_cls, box_ctr, box_delta, scores, boxes = outs

    shifts = jnp.tile(shift_img, (n, 1))
    return images_norm, box_cls, box_ctr, box_delta, scores, boxes, shifts
